# baseline (speedup 1.0000x reference)
import functools

import jax
import jax.numpy as jnp
from jax import lax
from jax.experimental import pallas as pl
from jax.experimental.pallas import tpu as pltpu
from jax.experimental.pallas import tpu_sc as plsc

B, C, H, W = 4, 192, 224, 224
CB = 32
NCB = C // CB
K = 16
NCHUNK = C // 16
BIG = 1 << 30


def _copy_mag_kernel(x_ref, y_ref, mag_ref, acc_ref):
    cb = pl.program_id(1)
    x = x_ref[...]
    y_ref[...] = x
    acc_ref[pl.ds(cb, 1), :] = jnp.sum(x * x, axis=(-1, -2))

    @pl.when(cb == NCB - 1)
    def _():
        mag_ref[...] = acc_ref[...][None]


def _sc_select_kernel(mags_hbm, idx_hbm, mags_v, idx_v):
    c = lax.axis_index("c")
    s = lax.axis_index("s")

    def _minsplat(v):
        for _ in range(4):
            sv = lax.sort(v, dimension=0)
            v = jnp.minimum(sv, lax.rev(sv, (0,)))
        return v

    @pl.when((c == 0) & (s < B))
    def _():
        b = s
        pltpu.sync_copy(mags_hbm.at[b], mags_v)
        lanes = lax.iota(jnp.int32, 16)
        vs = [mags_v[pl.ds(16 * j, 16)] for j in range(NCHUNK)]
        m16 = vs[0]
        for j in range(1, NCHUNK):
            m16 = jnp.minimum(m16, vs[j])
        minsplat = _minsplat(m16)
        cands = [jnp.where(vs[j] <= minsplat, lanes + 16 * j, BIG)
                 for j in range(NCHUNK)]
        acc = lax.sort(cands[0], dimension=0)
        for j in range(1, NCHUNK):
            sj = lax.sort(cands[j], dimension=0)
            acc = lax.sort(jnp.minimum(acc, lax.rev(sj, (0,))), dimension=0)
        firstsplat = _minsplat(acc)
        idx_v[...] = jnp.where(acc >= BIG, firstsplat, acc)
        pltpu.sync_copy(idx_v, idx_hbm.at[b])


_sc_select = pl.kernel(
    _sc_select_kernel,
    out_type=jax.ShapeDtypeStruct((B, 16), jnp.int32),
    mesh=plsc.VectorSubcoreMesh(core_axis_name="c", subcore_axis_name="s"),
    scratch_types=[
        pltpu.VMEM((C,), jnp.float32),
        pltpu.VMEM((16,), jnp.int32),
    ],
    compiler_params=pltpu.CompilerParams(needs_layout_passes=False),
)


def _zero_kernel(idx_ref, y_in_ref, y_out_ref):
    del idx_ref, y_in_ref
    y_out_ref[...] = jnp.zeros_like(y_out_ref)


@jax.jit
def kernel(input):
    y1, mag = pl.pallas_call(
        _copy_mag_kernel,
        grid=(B, NCB),
        in_specs=[pl.BlockSpec((1, CB, H, W), lambda b, cb: (b, cb, 0, 0))],
        out_specs=[
            pl.BlockSpec((1, CB, H, W), lambda b, cb: (b, cb, 0, 0)),
            pl.BlockSpec((1, NCB, CB), lambda b, cb: (b, 0, 0)),
        ],
        out_shape=[
            jax.ShapeDtypeStruct((B, C, H, W), input.dtype),
            jax.ShapeDtypeStruct((B, NCB, CB), jnp.float32),
        ],
        scratch_shapes=[pltpu.VMEM((NCB, CB), jnp.float32)],
    )(input)

    idx = _sc_select(mag.reshape(B, C))

    grid_spec = pltpu.PrefetchScalarGridSpec(
        num_scalar_prefetch=1,
        grid=(B, K),
        in_specs=[
            pl.BlockSpec((1, 1, H, W), lambda b, k, idx: (b, idx[b, k], 0, 0)),
        ],
        out_specs=pl.BlockSpec(
            (1, 1, H, W), lambda b, k, idx: (b, idx[b, k], 0, 0)),
    )
    y = pl.pallas_call(
        _zero_kernel,
        grid_spec=grid_spec,
        out_shape=jax.ShapeDtypeStruct((B, C, H, W), input.dtype),
        input_output_aliases={1: 0},
    )(idx, y1)
    return y


# speedup vs baseline: 1.0117x; 1.0117x over previous
import functools

import jax
import jax.numpy as jnp
from jax import lax
from jax.experimental import pallas as pl
from jax.experimental.pallas import tpu as pltpu
from jax.experimental.pallas import tpu_sc as plsc

B, C, H, W = 4, 192, 224, 224
CB = 32
NCB = C // CB
K = 16
NCHUNK = C // 16
BIG = 1 << 30


def _copy_mag_kernel(x_ref, y_ref, mag_ref, acc_ref):
    cb = pl.program_id(1)
    x = x_ref[...]
    y_ref[...] = x
    acc_ref[pl.ds(cb, 1), :] = jnp.sum(x * x, axis=(-1, -2))

    @pl.when(cb == NCB - 1)
    def _():
        mag_ref[...] = acc_ref[...][None]


def _sc_select_kernel(mags_hbm, idx_hbm, mags_v, idx_v):
    c = lax.axis_index("c")
    s = lax.axis_index("s")

    def _minsplat(v):
        for _ in range(4):
            sv = lax.sort(v, dimension=0)
            v = jnp.minimum(sv, lax.rev(sv, (0,)))
        return v

    @pl.when((c == 0) & (s < B))
    def _():
        b = s
        pltpu.sync_copy(mags_hbm.at[b], mags_v)
        lanes = lax.iota(jnp.int32, 16)
        vs = [mags_v[(16 * j) // CB, pl.ds((16 * j) % CB, 16)]
              for j in range(NCHUNK)]
        m16 = vs[0]
        for j in range(1, NCHUNK):
            m16 = jnp.minimum(m16, vs[j])
        minsplat = _minsplat(m16)
        cands = [jnp.where(vs[j] <= minsplat, lanes + 16 * j, BIG)
                 for j in range(NCHUNK)]
        acc = lax.sort(cands[0], dimension=0)
        for j in range(1, NCHUNK):
            sj = lax.sort(cands[j], dimension=0)
            acc = lax.sort(jnp.minimum(acc, lax.rev(sj, (0,))), dimension=0)
        firstsplat = _minsplat(acc)
        idx_v[...] = jnp.where(acc >= BIG, firstsplat, acc)
        pltpu.sync_copy(idx_v, idx_hbm.at[b])


_sc_select = pl.kernel(
    _sc_select_kernel,
    out_type=jax.ShapeDtypeStruct((B, 16), jnp.int32),
    mesh=plsc.VectorSubcoreMesh(core_axis_name="c", subcore_axis_name="s"),
    scratch_types=[
        pltpu.VMEM((NCB, CB), jnp.float32),
        pltpu.VMEM((16,), jnp.int32),
    ],
    compiler_params=pltpu.CompilerParams(needs_layout_passes=False),
)


def _zero_kernel(idx_ref, y_in_ref, y_out_ref):
    del idx_ref, y_in_ref
    y_out_ref[...] = jnp.zeros_like(y_out_ref)


@jax.jit
def kernel(input):
    y1, mag = pl.pallas_call(
        _copy_mag_kernel,
        grid=(B, NCB),
        in_specs=[pl.BlockSpec((1, CB, H, W), lambda b, cb: (b, cb, 0, 0))],
        out_specs=[
            pl.BlockSpec((1, CB, H, W), lambda b, cb: (b, cb, 0, 0)),
            pl.BlockSpec((1, NCB, CB), lambda b, cb: (b, 0, 0)),
        ],
        out_shape=[
            jax.ShapeDtypeStruct((B, C, H, W), input.dtype),
            jax.ShapeDtypeStruct((B, NCB, CB), jnp.float32),
        ],
        scratch_shapes=[pltpu.VMEM((NCB, CB), jnp.float32)],
    )(input)

    idx = _sc_select(mag)

    grid_spec = pltpu.PrefetchScalarGridSpec(
        num_scalar_prefetch=1,
        grid=(B, K),
        in_specs=[
            pl.BlockSpec((1, 1, H, W), lambda b, k, idx: (b, idx[b, k], 0, 0)),
        ],
        out_specs=pl.BlockSpec(
            (1, 1, H, W), lambda b, k, idx: (b, idx[b, k], 0, 0)),
    )
    y = pl.pallas_call(
        _zero_kernel,
        grid_spec=grid_spec,
        out_shape=jax.ShapeDtypeStruct((B, C, H, W), input.dtype),
        input_output_aliases={1: 0},
    )(idx, y1)
    return y


# speedup vs baseline: 1.0229x; 1.0111x over previous
import functools

import jax
import jax.numpy as jnp
from jax import lax
from jax.experimental import pallas as pl
from jax.experimental.pallas import tpu as pltpu
from jax.experimental.pallas import tpu_sc as plsc

B, C, H, W = 4, 192, 224, 224
CB = 32
NCB = C // CB
K = 8
NCHUNK = C // 16
BIG = 1 << 30


def _copy_mag_kernel(x_ref, y_ref, mag_ref, acc_ref):
    cb = pl.program_id(1)
    x = x_ref[...]
    y_ref[...] = x
    acc_ref[pl.ds(cb, 1), :] = jnp.sum(x * x, axis=(-1, -2))

    @pl.when(cb == NCB - 1)
    def _():
        mag_ref[...] = acc_ref[...][None]


def _sc_select_kernel(mags_hbm, idx_hbm, mags_v, idx_v):
    c = lax.axis_index("c")
    s = lax.axis_index("s")

    def _minsplat(v):
        for _ in range(4):
            sv = lax.sort(v, dimension=0)
            v = jnp.minimum(sv, lax.rev(sv, (0,)))
        return v

    @pl.when((c == 0) & (s < B))
    def _():
        b = s
        pltpu.sync_copy(mags_hbm.at[b], mags_v)
        lanes = lax.iota(jnp.int32, 16)
        vs = [mags_v[(16 * j) // CB, pl.ds((16 * j) % CB, 16)]
              for j in range(NCHUNK)]
        m16 = vs[0]
        for j in range(1, NCHUNK):
            m16 = jnp.minimum(m16, vs[j])
        minsplat = _minsplat(m16)
        cands = [jnp.where(vs[j] <= minsplat, lanes + 16 * j, BIG)
                 for j in range(NCHUNK)]
        acc = lax.sort(cands[0], dimension=0)
        for j in range(1, NCHUNK):
            sj = lax.sort(cands[j], dimension=0)
            acc = lax.sort(jnp.minimum(acc, lax.rev(sj, (0,))), dimension=0)
        firstsplat = _minsplat(acc)
        idx_v[...] = jnp.where(acc >= BIG, firstsplat, acc)
        pltpu.sync_copy(idx_v, idx_hbm.at[b])


_sc_select = pl.kernel(
    _sc_select_kernel,
    out_type=jax.ShapeDtypeStruct((B, 16), jnp.int32),
    mesh=plsc.VectorSubcoreMesh(core_axis_name="c", subcore_axis_name="s"),
    scratch_types=[
        pltpu.VMEM((NCB, CB), jnp.float32),
        pltpu.VMEM((16,), jnp.int32),
    ],
    compiler_params=pltpu.CompilerParams(needs_layout_passes=False),
)


def _zero_kernel(idx_ref, y_in_ref, y_out_ref):
    del idx_ref, y_in_ref
    y_out_ref[...] = jnp.zeros_like(y_out_ref)


@jax.jit
def kernel(input):
    y1, mag = pl.pallas_call(
        _copy_mag_kernel,
        grid=(B, NCB),
        in_specs=[pl.BlockSpec((1, CB, H, W), lambda b, cb: (b, cb, 0, 0))],
        out_specs=[
            pl.BlockSpec((1, CB, H, W), lambda b, cb: (b, cb, 0, 0)),
            pl.BlockSpec((1, NCB, CB), lambda b, cb: (b, 0, 0)),
        ],
        out_shape=[
            jax.ShapeDtypeStruct((B, C, H, W), input.dtype),
            jax.ShapeDtypeStruct((B, NCB, CB), jnp.float32),
        ],
        scratch_shapes=[pltpu.VMEM((NCB, CB), jnp.float32)],
    )(input)

    idx = _sc_select(mag)

    grid_spec = pltpu.PrefetchScalarGridSpec(
        num_scalar_prefetch=1,
        grid=(B, K),
        in_specs=[
            pl.BlockSpec((1, 1, H, W), lambda b, k, idx: (b, idx[b, k], 0, 0)),
        ],
        out_specs=pl.BlockSpec(
            (1, 1, H, W), lambda b, k, idx: (b, idx[b, k], 0, 0)),
    )
    y = pl.pallas_call(
        _zero_kernel,
        grid_spec=grid_spec,
        out_shape=jax.ShapeDtypeStruct((B, C, H, W), input.dtype),
        input_output_aliases={1: 0},
    )(idx, y1)
    return y
